# Initial kernel scaffold; baseline (speedup 1.0000x reference)
import functools

import jax
import jax.numpy as jnp
from jax import lax
from jax.experimental import pallas as pl
from jax.experimental.pallas import tpu as pltpu
from jax.experimental.pallas import tpu_sc as plsc

N = 10000
F = 128
HID = 128
C = 40
CP = 48
E = 320000

NC = 2
NS = 16
NW = NC * NS
CH = 128
NROW = 10016
RPT = NROW // NS
DUMP = 10008
EPW = (-(-E // NW) // CH + 1) * CH
NCHUNK = EPW // CH
EPAD = EPW * NW


def _mm1_body(x_ref, wl_ref, wr_ref, b1_ref, xl_ref, xr_ref):
    xb = x_ref[...]
    xl_ref[...] = jnp.dot(xb, wl_ref[...], preferred_element_type=jnp.float32)
    xr_ref[...] = (
        jnp.dot(xb, wr_ref[...], preferred_element_type=jnp.float32)
        + b1_ref[...]
    )


def _mm1(x, W1l, W1r, b1):
    bm = 1000
    return pl.pallas_call(
        _mm1_body,
        grid=(N // bm,),
        in_specs=[
            pl.BlockSpec((bm, F), lambda i: (i, 0)),
            pl.BlockSpec((F, HID), lambda i: (0, 0)),
            pl.BlockSpec((F, HID), lambda i: (0, 0)),
            pl.BlockSpec((1, HID), lambda i: (0, 0)),
        ],
        out_specs=[
            pl.BlockSpec((bm, HID), lambda i: (i, 0)),
            pl.BlockSpec((bm, HID), lambda i: (i, 0)),
        ],
        out_shape=[
            jax.ShapeDtypeStruct((N, HID), jnp.float32),
            jax.ShapeDtypeStruct((N, HID), jnp.float32),
        ],
    )(x, W1l, W1r, b1.reshape(1, HID))


def _sc_agg_body(width, with_deg, *refs):
    if with_deg:
        (tbl, srch, dsth, wh, zx, zd, aggp, degp,
         src_v, dst_v, w_v, rows_v, ones_v, accx, accd, sem) = refs
    else:
        (tbl, srch, dsth, wh, zx, aggp,
         src_v, dst_v, w_v, rows_v, accx, sem) = refs
    c = lax.axis_index("c")
    s = lax.axis_index("s")
    wid = s * NC + c
    r0 = s * RPT

    pltpu.sync_copy(zx.at[pl.ds(r0, RPT)], accx.at[pl.ds(r0, RPT)])
    if with_deg:
        pltpu.sync_copy(zd.at[pl.ds(r0, RPT)], accd.at[pl.ds(r0, RPT)])

        def init_ones(i, _):
            ones_v[i, :] = jnp.full((16,), 1.0, jnp.float32)
            return 0
        lax.fori_loop(0, CH, init_ones, 0)
    plsc.subcore_barrier()

    ngrp = width // 16
    base = wid * EPW

    def chunk(g, _):
        off = base + g * CH
        pltpu.sync_copy(srch.at[pl.ds(off, CH)], src_v)
        pltpu.sync_copy(dsth.at[pl.ds(off, CH)], dst_v)
        pltpu.sync_copy(wh.at[pl.ds(off, CH)], w_v)
        pltpu.async_copy(tbl.at[src_v], rows_v, sem).wait()

        def grp(q, _):
            wv = w_v[pl.ds(q * 16, 16)]
            for l in range(16):
                bw = lax.broadcast_in_dim(
                    lax.slice(wv, (l,), (l + 1,)), (16,), (0,))
                e = q * 16 + l
                for j in range(ngrp):
                    rows_v[e, pl.ds(j * 16, 16)] = (
                        rows_v[e, pl.ds(j * 16, 16)] * bw)
            return 0
        lax.fori_loop(0, CH // 16, grp, 0)

        pltpu.sync_copy(rows_v, accx.at[dst_v], add=True)
        if with_deg:
            pltpu.sync_copy(ones_v, accd.at[dst_v], add=True)
        return 0
    lax.fori_loop(0, NCHUNK, chunk, 0)
    plsc.subcore_barrier()

    pltpu.sync_copy(accx.at[pl.ds(r0, RPT)], aggp.at[c, pl.ds(r0, RPT)])
    if with_deg:
        pltpu.sync_copy(accd.at[pl.ds(r0, RPT)], degp.at[c, pl.ds(r0, RPT)])


def _sc_agg(width, with_deg):
    mesh = plsc.VectorSubcoreMesh(core_axis_name="c", subcore_axis_name="s")
    out_type = [jax.ShapeDtypeStruct((NC, NROW, width), jnp.float32)]
    scratch = [
        pltpu.VMEM((CH,), jnp.int32),
        pltpu.VMEM((CH,), jnp.int32),
        pltpu.VMEM((CH,), jnp.float32),
        pltpu.VMEM((CH, width), jnp.float32),
    ]
    if with_deg:
        out_type.append(jax.ShapeDtypeStruct((NC, NROW, 16), jnp.float32))
        scratch.append(pltpu.VMEM((CH, 16), jnp.float32))
    scratch.append(pltpu.VMEM_SHARED((NROW, width), jnp.float32))
    if with_deg:
        scratch.append(pltpu.VMEM_SHARED((NROW, 16), jnp.float32))
    scratch.append(pltpu.SemaphoreType.DMA)
    return pl.kernel(
        functools.partial(_sc_agg_body, width, with_deg),
        out_type=out_type,
        mesh=mesh,
        scratch_types=scratch,
    )


def _mid_body(a0_ref, a1_ref, d0_ref, d1_ref, xr_ref, wl_ref, wr_ref, b2_ref,
              hl_ref, hr_ref):
    agg = a0_ref[...] + a1_ref[...]
    deg = d0_ref[:, 0:1] + d1_ref[:, 0:1]
    rdeg = 1.0 / jnp.maximum(deg, 1.0)
    h = jnp.maximum(agg * rdeg + xr_ref[...], 0.0)
    hl_ref[...] = jnp.dot(h, wl_ref[...], preferred_element_type=jnp.float32)
    hr_ref[...] = (
        jnp.dot(h, wr_ref[...], preferred_element_type=jnp.float32)
        + b2_ref[...]
    )


def _mid(a0, a1, d0, d1, xr, W2lp, W2rp, b2p):
    bm = 1000
    return pl.pallas_call(
        _mid_body,
        grid=(N // bm,),
        in_specs=[
            pl.BlockSpec((bm, HID), lambda i: (i, 0)),
            pl.BlockSpec((bm, HID), lambda i: (i, 0)),
            pl.BlockSpec((bm, 16), lambda i: (i, 0)),
            pl.BlockSpec((bm, 16), lambda i: (i, 0)),
            pl.BlockSpec((bm, HID), lambda i: (i, 0)),
            pl.BlockSpec((HID, CP), lambda i: (0, 0)),
            pl.BlockSpec((HID, CP), lambda i: (0, 0)),
            pl.BlockSpec((1, CP), lambda i: (0, 0)),
        ],
        out_specs=[
            pl.BlockSpec((bm, CP), lambda i: (i, 0)),
            pl.BlockSpec((bm, CP), lambda i: (i, 0)),
        ],
        out_shape=[
            jax.ShapeDtypeStruct((N, CP), jnp.float32),
            jax.ShapeDtypeStruct((N, CP), jnp.float32),
        ],
    )(a0, a1, d0, d1, xr, W2lp, W2rp, b2p)


def _fin_body(a0_ref, a1_ref, d0_ref, d1_ref, hr_ref, out_ref):
    agg = a0_ref[...] + a1_ref[...]
    deg = d0_ref[:, 0:1] + d1_ref[:, 0:1]
    rdeg = 1.0 / jnp.maximum(deg, 1.0)
    logits = agg * rdeg + hr_ref[...]
    col = lax.broadcasted_iota(jnp.int32, logits.shape, 1)
    masked = jnp.where(col < C, logits, -1e30)
    m = jnp.max(masked, axis=1, keepdims=True)
    lse = jnp.log(jnp.sum(jnp.exp(masked - m), axis=1, keepdims=True)) + m
    out_ref[...] = logits - lse


def _fin(a0, a1, d0, d1, hr):
    bm = 1000
    return pl.pallas_call(
        _fin_body,
        grid=(N // bm,),
        in_specs=[
            pl.BlockSpec((bm, CP), lambda i: (i, 0)),
            pl.BlockSpec((bm, CP), lambda i: (i, 0)),
            pl.BlockSpec((bm, 16), lambda i: (i, 0)),
            pl.BlockSpec((bm, 16), lambda i: (i, 0)),
            pl.BlockSpec((bm, CP), lambda i: (i, 0)),
        ],
        out_specs=pl.BlockSpec((bm, CP), lambda i: (i, 0)),
        out_shape=jax.ShapeDtypeStruct((N, CP), jnp.float32),
    )(a0, a1, d0, d1, hr)


@jax.jit
def kernel(x, edge_index, edge_weight, W1l, W1r, b1, W2l, W2r, b2):
    src = edge_index[0]
    dst = edge_index[1]
    pad = EPAD - E
    srcp = jnp.concatenate([src, jnp.zeros((pad,), jnp.int32)])
    dstp = jnp.concatenate([dst, jnp.full((pad,), DUMP, jnp.int32)])
    wp = jnp.concatenate([edge_weight, jnp.zeros((pad,), jnp.float32)])

    zx = jnp.zeros((NROW, HID), jnp.float32)
    zd = jnp.zeros((NROW, 16), jnp.float32)
    zc = jnp.zeros((NROW, CP), jnp.float32)

    W2lp = jnp.pad(W2l, ((0, 0), (0, CP - C)))
    W2rp = jnp.pad(W2r, ((0, 0), (0, CP - C)))
    b2p = jnp.pad(b2, (0, CP - C)).reshape(1, CP)

    xl, xr = _mm1(x, W1l, W1r, b1)
    aggp, degp = _sc_agg(HID, True)(xl, srcp, dstp, wp, zx, zd)
    hl, hr = _mid(aggp[0], aggp[1], degp[0], degp[1], xr, W2lp, W2rp, b2p)
    (agg2p,) = _sc_agg(CP, False)(hl, srcp, dstp, wp, zc)
    out = _fin(agg2p[0], agg2p[1], degp[0], degp[1], hr)
    return out[:, :C]


# speedup vs baseline: 4.2589x; 4.2589x over previous
import functools

import jax
import jax.numpy as jnp
from jax import lax
from jax.experimental import pallas as pl
from jax.experimental.pallas import tpu as pltpu
from jax.experimental.pallas import tpu_sc as plsc

N = 10000
F = 128
HID = 128
C = 40
CP = 48
E = 320000

NC = 2
NS = 16
NW = NC * NS
CH = 128
NROW = 10112
RPT = NROW // NS
DUMP = 10008
EPW = (-(-E // NW) // CH + 1) * CH
NCHUNK = EPW // CH
EPAD = EPW * NW


def _mm1_body(x_ref, wl_ref, wr_ref, b1_ref, xl_ref, xr_ref):
    xb = x_ref[...]
    xl_ref[...] = jnp.dot(xb, wl_ref[...], preferred_element_type=jnp.float32)
    xr_ref[...] = (
        jnp.dot(xb, wr_ref[...], preferred_element_type=jnp.float32)
        + b1_ref[...]
    )


def _mm1(x, W1l, W1r, b1):
    bm = 1000
    return pl.pallas_call(
        _mm1_body,
        grid=(N // bm,),
        in_specs=[
            pl.BlockSpec((bm, F), lambda i: (i, 0)),
            pl.BlockSpec((F, HID), lambda i: (0, 0)),
            pl.BlockSpec((F, HID), lambda i: (0, 0)),
            pl.BlockSpec((1, HID), lambda i: (0, 0)),
        ],
        out_specs=[
            pl.BlockSpec((bm, HID), lambda i: (i, 0)),
            pl.BlockSpec((bm, HID), lambda i: (i, 0)),
        ],
        out_shape=[
            jax.ShapeDtypeStruct((N, HID), jnp.float32),
            jax.ShapeDtypeStruct((N, HID), jnp.float32),
        ],
    )(x, W1l, W1r, b1.reshape(1, HID))


def _sc_agg_body(width, with_deg, *refs):
    if with_deg:
        (tbl, srch, dsth, wh, zx, zd, aggp, degp,
         src_v, dst_v, w_v, rows_v, ones_v, accx, accd, sem) = refs
    else:
        (tbl, srch, dsth, wh, zx, aggp,
         src_v, dst_v, w_v, rows_v, accx, sem) = refs
    c = lax.axis_index("c")
    s = lax.axis_index("s")
    wid = s * NC + c
    r0 = pl.multiple_of(s * RPT, 8)

    pltpu.sync_copy(zx.at[pl.ds(r0, RPT)], accx.at[pl.ds(r0, RPT)])
    if with_deg:
        pltpu.sync_copy(zd.at[pl.ds(r0, RPT)], accd.at[pl.ds(r0, RPT)])

        def init_ones(i, _):
            ones_v[i, :] = jnp.full((16,), 1.0, jnp.float32)
            return 0
        lax.fori_loop(0, CH, init_ones, 0)
    plsc.subcore_barrier()

    ngrp = width // 16
    base = wid * EPW

    def chunk(g, _):
        off = base + g * CH
        pltpu.sync_copy(srch.at[pl.ds(off, CH)], src_v)
        pltpu.sync_copy(dsth.at[pl.ds(off, CH)], dst_v)
        pltpu.sync_copy(wh.at[pl.ds(off, CH)], w_v)
        pltpu.async_copy(tbl.at[src_v], rows_v, sem).wait()

        def grp(q, _):
            wv = w_v[pl.ds(q * 16, 16)]
            for l in range(16):
                bw = lax.broadcast_in_dim(
                    lax.slice(wv, (l,), (l + 1,)), (16,), (0,))
                e = q * 16 + l
                for j in range(ngrp):
                    rows_v[e, pl.ds(j * 16, 16)] = (
                        rows_v[e, pl.ds(j * 16, 16)] * bw)
            return 0
        lax.fori_loop(0, CH // 16, grp, 0)

        pltpu.sync_copy(rows_v, accx.at[dst_v], add=True)
        if with_deg:
            pltpu.sync_copy(ones_v, accd.at[dst_v], add=True)
        return 0
    lax.fori_loop(0, NCHUNK, chunk, 0)
    plsc.subcore_barrier()

    pltpu.sync_copy(accx.at[pl.ds(r0, RPT)], aggp.at[c, pl.ds(r0, RPT)])
    if with_deg:
        pltpu.sync_copy(accd.at[pl.ds(r0, RPT)], degp.at[c, pl.ds(r0, RPT)])


def _sc_agg(width, with_deg):
    mesh = plsc.VectorSubcoreMesh(core_axis_name="c", subcore_axis_name="s")
    out_type = [jax.ShapeDtypeStruct((NC, NROW, width), jnp.float32)]
    scratch = [
        pltpu.VMEM((CH,), jnp.int32),
        pltpu.VMEM((CH,), jnp.int32),
        pltpu.VMEM((CH,), jnp.float32),
        pltpu.VMEM((CH, width), jnp.float32),
    ]
    if with_deg:
        out_type.append(jax.ShapeDtypeStruct((NC, NROW, 16), jnp.float32))
        scratch.append(pltpu.VMEM((CH, 16), jnp.float32))
    scratch.append(pltpu.VMEM_SHARED((NROW, width), jnp.float32))
    if with_deg:
        scratch.append(pltpu.VMEM_SHARED((NROW, 16), jnp.float32))
    scratch.append(pltpu.SemaphoreType.DMA)
    return pl.kernel(
        functools.partial(_sc_agg_body, width, with_deg),
        out_type=out_type,
        mesh=mesh,
        scratch_types=scratch,
        compiler_params=pltpu.CompilerParams(use_tc_tiling_on_sc=False),
    )


def _mid_body(a0_ref, a1_ref, d0_ref, d1_ref, xr_ref, wl_ref, wr_ref, b2_ref,
              hl_ref, hr_ref):
    agg = a0_ref[...] + a1_ref[...]
    deg = d0_ref[:, 0:1] + d1_ref[:, 0:1]
    rdeg = 1.0 / jnp.maximum(deg, 1.0)
    h = jnp.maximum(agg * rdeg + xr_ref[...], 0.0)
    hl_ref[...] = jnp.dot(h, wl_ref[...], preferred_element_type=jnp.float32)
    hr_ref[...] = (
        jnp.dot(h, wr_ref[...], preferred_element_type=jnp.float32)
        + b2_ref[...]
    )


def _mid(a0, a1, d0, d1, xr, W2lp, W2rp, b2p):
    bm = 1000
    return pl.pallas_call(
        _mid_body,
        grid=(N // bm,),
        in_specs=[
            pl.BlockSpec((bm, HID), lambda i: (i, 0)),
            pl.BlockSpec((bm, HID), lambda i: (i, 0)),
            pl.BlockSpec((bm, 16), lambda i: (i, 0)),
            pl.BlockSpec((bm, 16), lambda i: (i, 0)),
            pl.BlockSpec((bm, HID), lambda i: (i, 0)),
            pl.BlockSpec((HID, CP), lambda i: (0, 0)),
            pl.BlockSpec((HID, CP), lambda i: (0, 0)),
            pl.BlockSpec((1, CP), lambda i: (0, 0)),
        ],
        out_specs=[
            pl.BlockSpec((bm, CP), lambda i: (i, 0)),
            pl.BlockSpec((bm, CP), lambda i: (i, 0)),
        ],
        out_shape=[
            jax.ShapeDtypeStruct((N, CP), jnp.float32),
            jax.ShapeDtypeStruct((N, CP), jnp.float32),
        ],
    )(a0, a1, d0, d1, xr, W2lp, W2rp, b2p)


def _fin_body(a0_ref, a1_ref, d0_ref, d1_ref, hr_ref, out_ref):
    agg = a0_ref[...] + a1_ref[...]
    deg = d0_ref[:, 0:1] + d1_ref[:, 0:1]
    rdeg = 1.0 / jnp.maximum(deg, 1.0)
    logits = agg * rdeg + hr_ref[...]
    col = lax.broadcasted_iota(jnp.int32, logits.shape, 1)
    masked = jnp.where(col < C, logits, -1e30)
    m = jnp.max(masked, axis=1, keepdims=True)
    lse = jnp.log(jnp.sum(jnp.exp(masked - m), axis=1, keepdims=True)) + m
    out_ref[...] = logits - lse


def _fin(a0, a1, d0, d1, hr):
    bm = 1000
    return pl.pallas_call(
        _fin_body,
        grid=(N // bm,),
        in_specs=[
            pl.BlockSpec((bm, CP), lambda i: (i, 0)),
            pl.BlockSpec((bm, CP), lambda i: (i, 0)),
            pl.BlockSpec((bm, 16), lambda i: (i, 0)),
            pl.BlockSpec((bm, 16), lambda i: (i, 0)),
            pl.BlockSpec((bm, CP), lambda i: (i, 0)),
        ],
        out_specs=pl.BlockSpec((bm, CP), lambda i: (i, 0)),
        out_shape=jax.ShapeDtypeStruct((N, CP), jnp.float32),
    )(a0, a1, d0, d1, hr)


@jax.jit
def kernel(x, edge_index, edge_weight, W1l, W1r, b1, W2l, W2r, b2):
    src = edge_index[0]
    dst = edge_index[1]
    pad = EPAD - E
    srcp = jnp.concatenate([src, jnp.zeros((pad,), jnp.int32)])
    dstp = jnp.concatenate([dst, jnp.full((pad,), DUMP, jnp.int32)])
    wp = jnp.concatenate([edge_weight, jnp.zeros((pad,), jnp.float32)])

    zx = jnp.zeros((NROW, HID), jnp.float32)
    zd = jnp.zeros((NROW, 16), jnp.float32)
    zc = jnp.zeros((NROW, CP), jnp.float32)

    W2lp = jnp.pad(W2l, ((0, 0), (0, CP - C)))
    W2rp = jnp.pad(W2r, ((0, 0), (0, CP - C)))
    b2p = jnp.pad(b2, (0, CP - C)).reshape(1, CP)

    xl, xr = _mm1(x, W1l, W1r, b1)
    aggp, degp = _sc_agg(HID, True)(xl, srcp, dstp, wp, zx, zd)
    hl, hr = _mid(aggp[0], aggp[1], degp[0], degp[1], xr, W2lp, W2rp, b2p)
    (agg2p,) = _sc_agg(CP, False)(hl, srcp, dstp, wp, zc)
    out = _fin(agg2p[0], agg2p[1], degp[0], degp[1], hr)
    return out[:, :C]


# speedup vs baseline: 5.1701x; 1.2139x over previous
import functools

import jax
import jax.numpy as jnp
from jax import lax
from jax.experimental import pallas as pl
from jax.experimental.pallas import tpu as pltpu
from jax.experimental.pallas import tpu_sc as plsc

N = 10000
F = 128
HID = 128
C = 40
CP = 48
E = 320000

NC = 2
NS = 16
NW = NC * NS
CH = 128
NROW = 10112
RPT = NROW // NS
DUMP0 = 10048
EPW = 10240
NCHUNK = EPW // CH
IBLK = 8
EPAD = EPW * NW
ERW = E // NW


def _mm1_body(x_ref, wl_ref, wr_ref, b1_ref, xl_ref, xr_ref):
    xb = x_ref[...]
    xl_ref[...] = jnp.dot(xb, wl_ref[...], preferred_element_type=jnp.float32)
    xr_ref[...] = (
        jnp.dot(xb, wr_ref[...], preferred_element_type=jnp.float32)
        + b1_ref[...]
    )


def _mm1(x, W1l, W1r, b1):
    bm = 1000
    return pl.pallas_call(
        _mm1_body,
        grid=(N // bm,),
        in_specs=[
            pl.BlockSpec((bm, F), lambda i: (i, 0)),
            pl.BlockSpec((F, HID), lambda i: (0, 0)),
            pl.BlockSpec((F, HID), lambda i: (0, 0)),
            pl.BlockSpec((1, HID), lambda i: (0, 0)),
        ],
        out_specs=[
            pl.BlockSpec((bm, HID), lambda i: (i, 0)),
            pl.BlockSpec((bm, HID), lambda i: (i, 0)),
        ],
        out_shape=[
            jax.ShapeDtypeStruct((N, HID), jnp.float32),
            jax.ShapeDtypeStruct((N, HID), jnp.float32),
        ],
    )(x, W1l, W1r, b1.reshape(1, HID))


def _sc_agg_body(width, with_deg, *refs):
    if with_deg:
        (tbl, srch, dsth, wh, zx, zd, aggp, degp,
         src_v, dst_v, w_v, rows0, rows1, ones_v, accx, accd,
         sem0, sem1) = refs
    else:
        (tbl, srch, dsth, wh, zx, aggp,
         src_v, dst_v, w_v, rows0, rows1, accx, sem0, sem1) = refs
    c = lax.axis_index("c")
    s = lax.axis_index("s")
    wid = s * NC + c
    r0 = pl.multiple_of(s * RPT, 8)

    pltpu.sync_copy(zx.at[pl.ds(r0, RPT)], accx.at[pl.ds(r0, RPT)])
    if with_deg:
        pltpu.sync_copy(zd.at[pl.ds(r0, RPT)], accd.at[pl.ds(r0, RPT)])

        def init_ones(i, _):
            ones_v[i, :] = jnp.full((16,), 1.0, jnp.float32)
            return 0
        lax.fori_loop(0, CH, init_ones, 0)

    plsc.subcore_barrier()

    ngrp = width // 16

    def proc(g, rows_v):
        def grp(q, _):
            wv = w_v[g, pl.ds(q * 16, 16)]
            for l in range(16):
                bw = lax.broadcast_in_dim(
                    lax.slice(wv, (l,), (l + 1,)), (16,), (0,))
                e = q * 16 + l
                for j in range(ngrp):
                    rows_v[e, pl.ds(j * 16, 16)] = (
                        rows_v[e, pl.ds(j * 16, 16)] * bw)
            return 0
        lax.fori_loop(0, CH // 16, grp, 0)
        pltpu.sync_copy(rows_v, accx.at[dst_v.at[g]], add=True)
        if with_deg:
            pltpu.sync_copy(ones_v, accd.at[dst_v.at[g]], add=True)

    def block(b, _):
        crow = wid * NCHUNK + b * IBLK
        pltpu.sync_copy(srch.at[pl.ds(crow, IBLK)], src_v)
        pltpu.sync_copy(dsth.at[pl.ds(crow, IBLK)], dst_v)
        pltpu.sync_copy(wh.at[pl.ds(crow, IBLK)], w_v)
        pltpu.async_copy(tbl.at[src_v.at[0]], rows0, sem0)

        def pair(i, _):
            g0 = i * 2
            pltpu.async_copy(tbl.at[src_v.at[g0 + 1]], rows1, sem1)
            pltpu.make_async_copy(tbl.at[src_v.at[g0]], rows0, sem0).wait()
            proc(g0, rows0)

            @pl.when(g0 + 2 < IBLK)
            def _():
                pltpu.async_copy(tbl.at[src_v.at[g0 + 2]], rows0, sem0)
            pltpu.make_async_copy(
                tbl.at[src_v.at[g0 + 1]], rows1, sem1).wait()
            proc(g0 + 1, rows1)
            return 0
        lax.fori_loop(0, IBLK // 2, pair, 0)
        return 0
    lax.fori_loop(0, NCHUNK // IBLK, block, 0)
    plsc.subcore_barrier()

    pltpu.sync_copy(accx.at[pl.ds(r0, RPT)], aggp.at[c, pl.ds(r0, RPT)])
    if with_deg:
        pltpu.sync_copy(accd.at[pl.ds(r0, RPT)], degp.at[c, pl.ds(r0, RPT)])


def _sc_agg(width, with_deg):
    mesh = plsc.VectorSubcoreMesh(core_axis_name="c", subcore_axis_name="s")
    out_type = [jax.ShapeDtypeStruct((NC, NROW, width), jnp.float32)]
    scratch = [
        pltpu.VMEM((IBLK, CH), jnp.int32),
        pltpu.VMEM((IBLK, CH), jnp.int32),
        pltpu.VMEM((IBLK, CH), jnp.float32),
        pltpu.VMEM((CH, width), jnp.float32),
        pltpu.VMEM((CH, width), jnp.float32),
    ]
    if with_deg:
        out_type.append(jax.ShapeDtypeStruct((NC, NROW, 16), jnp.float32))
        scratch.append(pltpu.VMEM((CH, 16), jnp.float32))
    scratch.append(pltpu.VMEM_SHARED((NROW, width), jnp.float32))
    if with_deg:
        scratch.append(pltpu.VMEM_SHARED((NROW, 16), jnp.float32))
    scratch.append(pltpu.SemaphoreType.DMA)
    scratch.append(pltpu.SemaphoreType.DMA)
    return pl.kernel(
        functools.partial(_sc_agg_body, width, with_deg),
        out_type=out_type,
        mesh=mesh,
        scratch_types=scratch,
        compiler_params=pltpu.CompilerParams(use_tc_tiling_on_sc=False),
    )


def _mid_body(a0_ref, a1_ref, d0_ref, d1_ref, xr_ref, wl_ref, wr_ref, b2_ref,
              hl_ref, hr_ref):
    agg = a0_ref[...] + a1_ref[...]
    deg = d0_ref[:, 0:1] + d1_ref[:, 0:1]
    rdeg = 1.0 / jnp.maximum(deg, 1.0)
    h = jnp.maximum(agg * rdeg + xr_ref[...], 0.0)
    hl_ref[...] = jnp.dot(h, wl_ref[...], preferred_element_type=jnp.float32)
    hr_ref[...] = (
        jnp.dot(h, wr_ref[...], preferred_element_type=jnp.float32)
        + b2_ref[...]
    )


def _mid(a0, a1, d0, d1, xr, W2lp, W2rp, b2p):
    bm = 1000
    return pl.pallas_call(
        _mid_body,
        grid=(N // bm,),
        in_specs=[
            pl.BlockSpec((bm, HID), lambda i: (i, 0)),
            pl.BlockSpec((bm, HID), lambda i: (i, 0)),
            pl.BlockSpec((bm, 16), lambda i: (i, 0)),
            pl.BlockSpec((bm, 16), lambda i: (i, 0)),
            pl.BlockSpec((bm, HID), lambda i: (i, 0)),
            pl.BlockSpec((HID, CP), lambda i: (0, 0)),
            pl.BlockSpec((HID, CP), lambda i: (0, 0)),
            pl.BlockSpec((1, CP), lambda i: (0, 0)),
        ],
        out_specs=[
            pl.BlockSpec((bm, CP), lambda i: (i, 0)),
            pl.BlockSpec((bm, CP), lambda i: (i, 0)),
        ],
        out_shape=[
            jax.ShapeDtypeStruct((N, CP), jnp.float32),
            jax.ShapeDtypeStruct((N, CP), jnp.float32),
        ],
    )(a0, a1, d0, d1, xr, W2lp, W2rp, b2p)


def _fin_body(a0_ref, a1_ref, d0_ref, d1_ref, hr_ref, out_ref):
    agg = a0_ref[...] + a1_ref[...]
    deg = d0_ref[:, 0:1] + d1_ref[:, 0:1]
    rdeg = 1.0 / jnp.maximum(deg, 1.0)
    logits = agg * rdeg + hr_ref[...]
    col = lax.broadcasted_iota(jnp.int32, logits.shape, 1)
    masked = jnp.where(col < C, logits, -1e30)
    m = jnp.max(masked, axis=1, keepdims=True)
    lse = jnp.log(jnp.sum(jnp.exp(masked - m), axis=1, keepdims=True)) + m
    out_ref[...] = logits - lse


def _fin(a0, a1, d0, d1, hr):
    bm = 1000
    return pl.pallas_call(
        _fin_body,
        grid=(N // bm,),
        in_specs=[
            pl.BlockSpec((bm, CP), lambda i: (i, 0)),
            pl.BlockSpec((bm, CP), lambda i: (i, 0)),
            pl.BlockSpec((bm, 16), lambda i: (i, 0)),
            pl.BlockSpec((bm, 16), lambda i: (i, 0)),
            pl.BlockSpec((bm, CP), lambda i: (i, 0)),
        ],
        out_specs=pl.BlockSpec((bm, CP), lambda i: (i, 0)),
        out_shape=jax.ShapeDtypeStruct((N, CP), jnp.float32),
    )(a0, a1, d0, d1, hr)


@jax.jit
def kernel(x, edge_index, edge_weight, W1l, W1r, b1, W2l, W2r, b2):
    src = edge_index[0]
    dst = edge_index[1]
    padw = EPW - ERW
    dpad = jnp.broadcast_to(
        DUMP0 + (jnp.arange(padw, dtype=jnp.int32) % 64), (NW, padw))
    srcp = jnp.concatenate(
        [src.reshape(NW, ERW), jnp.zeros((NW, padw), jnp.int32)],
        axis=1).reshape(EPAD // CH, CH)
    dstp = jnp.concatenate(
        [dst.reshape(NW, ERW), dpad], axis=1).reshape(EPAD // CH, CH)
    wp = jnp.concatenate(
        [edge_weight.reshape(NW, ERW), jnp.zeros((NW, padw), jnp.float32)],
        axis=1).reshape(EPAD // CH, CH)

    zx = jnp.zeros((NROW, HID), jnp.float32)
    zd = jnp.zeros((NROW, 16), jnp.float32)
    zc = jnp.zeros((NROW, CP), jnp.float32)

    W2lp = jnp.pad(W2l, ((0, 0), (0, CP - C)))
    W2rp = jnp.pad(W2r, ((0, 0), (0, CP - C)))
    b2p = jnp.pad(b2, (0, CP - C)).reshape(1, CP)

    xl, xr = _mm1(x, W1l, W1r, b1)
    aggp, degp = _sc_agg(HID, True)(xl, srcp, dstp, wp, zx, zd)
    hl, hr = _mid(aggp[0], aggp[1], degp[0], degp[1], xr, W2lp, W2rp, b2p)
    (agg2p,) = _sc_agg(CP, False)(hl, srcp, dstp, wp, zc)
    out = _fin(agg2p[0], agg2p[1], degp[0], degp[1], hr)
    return out[:, :C]
